# baseline (device time: 67987 ns/iter reference)
import jax
import jax.numpy as jnp
from jax import lax
from jax.experimental import pallas as pl
from jax.experimental.pallas import tpu as pltpu

N_DEV = 8
B, SQ, SKV = 2, 256, 256
HQ_LOCAL, DH = 4, 64
HD = HQ_LOCAL * DH
DM = 512
BLK = 64


def kernel(x, Wq, K_ext, V_ext, Wo):
    K2 = K_ext.reshape(B, SKV, -1)
    V2 = V_ext.reshape(B, SKV, -1)

    def body(x_ref, wq_ref, k_ref, v_ref, wo_ref, out_ref,
             comm_ref, send_sems, recv_sems):
        my = lax.axis_index("i")
        left = lax.rem(my + N_DEV - 1, N_DEV)
        right = lax.rem(my + 1, N_DEV)

        barrier_sem = pltpu.get_barrier_semaphore()
        for nbr in (left, right):
            pl.semaphore_signal(barrier_sem, inc=1, device_id=(nbr,),
                                device_id_type=pl.DeviceIdType.MESH)
        pl.semaphore_wait(barrier_sem, 2)

        col0 = my * HD

        qb = lax.broadcasted_iota(jnp.int32, (SQ, SKV), 0) // BLK
        kb = lax.broadcasted_iota(jnp.int32, (SQ, SKV), 1) // BLK
        mask = (qb == kb) | (kb == 0) | ((qb + kb) % 3 == 0)

        for b in range(B):
            xb = x_ref[b].astype(jnp.bfloat16)
            q_all = jnp.dot(xb, wq_ref[:, :].astype(jnp.bfloat16),
                            preferred_element_type=jnp.float32)
            k_all = k_ref[b, :, pl.ds(col0, HD)].astype(jnp.bfloat16)
            v_all = v_ref[b, :, pl.ds(col0, HD)].astype(jnp.bfloat16)
            partial = jnp.zeros((SQ, DM), jnp.float32)
            for h in range(HQ_LOCAL):
                qh = q_all[:, h * DH:(h + 1) * DH].astype(jnp.bfloat16)
                kh = k_all[:, h * DH:(h + 1) * DH]
                vh = v_all[:, h * DH:(h + 1) * DH]
                scores = lax.dot_general(
                    qh, kh, (((1,), (1,)), ((), ())),
                    preferred_element_type=jnp.float32) * 0.125
                scores = jnp.where(mask, scores, -1e9)
                m = jnp.max(scores, axis=1, keepdims=True)
                w = jnp.exp(scores - m)
                w = w / jnp.sum(w, axis=1, keepdims=True)
                ctx = jnp.dot(w.astype(jnp.bfloat16), vh,
                              preferred_element_type=jnp.float32)
                partial = partial + jnp.dot(
                    ctx.astype(jnp.bfloat16),
                    wo_ref[h * DH:(h + 1) * DH, :].astype(jnp.bfloat16),
                    preferred_element_type=jnp.float32)
            out_ref[b, :, :] = partial
            comm_ref[0, b, :, :] = partial.astype(jnp.bfloat16)

        for hop in range(N_DEV - 1):
            rdma = pltpu.make_async_remote_copy(
                src_ref=comm_ref.at[hop],
                dst_ref=comm_ref.at[hop + 1],
                send_sem=send_sems.at[hop],
                recv_sem=recv_sems.at[hop + 1],
                device_id=(right,),
                device_id_type=pl.DeviceIdType.MESH,
            )
            rdma.start()
            rdma.wait()
            out_ref[:, :, :] = (out_ref[:, :, :]
                                + comm_ref[hop + 1].astype(jnp.float32))

    return pl.pallas_call(
        body,
        out_shape=jax.ShapeDtypeStruct((B, SQ, DM), jnp.float32),
        in_specs=[pl.BlockSpec(memory_space=pltpu.VMEM)] * 5,
        out_specs=pl.BlockSpec(memory_space=pltpu.VMEM),
        scratch_shapes=[
            pltpu.VMEM((N_DEV, B, SQ, DM), jnp.bfloat16),
            pltpu.SemaphoreType.DMA((N_DEV,)),
            pltpu.SemaphoreType.DMA((N_DEV,)),
        ],
        compiler_params=pltpu.CompilerParams(collective_id=0),
    )(x, Wq, K2, V2, Wo)


# device time: 27217 ns/iter; 2.4980x vs baseline; 2.4980x over previous
import jax
import jax.numpy as jnp
from jax import lax
from jax.experimental import pallas as pl
from jax.experimental.pallas import tpu as pltpu

N_DEV = 8
B, SQ, SKV = 2, 256, 256
HQ_LOCAL, DH = 4, 64
HD = HQ_LOCAL * DH
DM = 512
BLK = 64
ROWS = B * SQ
CH = ROWS // N_DEV


def kernel(x, Wq, K_ext, V_ext, Wo):
    K2 = K_ext.reshape(B, SKV, -1)
    V2 = V_ext.reshape(B, SKV, -1)

    def body(x_ref, wq_ref, k_ref, v_ref, wo_ref, out_ref,
             send_buf, recv_buf, gather_buf, bcast_buf,
             s1, r1, s2, r2):
        my = lax.axis_index("i")
        peers = [lax.rem(my + off, N_DEV) for off in range(1, N_DEV)]

        barrier_sem = pltpu.get_barrier_semaphore()
        for p in peers:
            pl.semaphore_signal(barrier_sem, inc=1, device_id=(p,),
                                device_id_type=pl.DeviceIdType.MESH)
        pl.semaphore_wait(barrier_sem, N_DEV - 1)

        col0 = my * HD

        qb = lax.broadcasted_iota(jnp.int32, (SQ, SKV), 0) // BLK
        kb = lax.broadcasted_iota(jnp.int32, (SQ, SKV), 1) // BLK
        mask = (qb == kb) | (kb == 0) | ((qb + kb) % 3 == 0)

        for b in range(B):
            xb = x_ref[b].astype(jnp.bfloat16)
            q_all = jnp.dot(xb, wq_ref[:, :].astype(jnp.bfloat16),
                            preferred_element_type=jnp.float32)
            k_all = k_ref[b, :, pl.ds(col0, HD)].astype(jnp.bfloat16)
            v_all = v_ref[b, :, pl.ds(col0, HD)].astype(jnp.bfloat16)
            partial = jnp.zeros((SQ, DM), jnp.float32)
            for h in range(HQ_LOCAL):
                qh = q_all[:, h * DH:(h + 1) * DH].astype(jnp.bfloat16)
                kh = k_all[:, h * DH:(h + 1) * DH]
                vh = v_all[:, h * DH:(h + 1) * DH]
                scores = lax.dot_general(
                    qh, kh, (((1,), (1,)), ((), ())),
                    preferred_element_type=jnp.float32) * 0.125
                scores = jnp.where(mask, scores, -1e9)
                m = jnp.max(scores, axis=1, keepdims=True)
                w = jnp.exp(scores - m)
                w = w / jnp.sum(w, axis=1, keepdims=True)
                ctx = jnp.dot(w.astype(jnp.bfloat16), vh,
                              preferred_element_type=jnp.float32)
                partial = partial + jnp.dot(
                    ctx.astype(jnp.bfloat16),
                    wo_ref[h * DH:(h + 1) * DH, :].astype(jnp.bfloat16),
                    preferred_element_type=jnp.float32)
            out_ref[pl.ds(b * SQ, SQ), :] = partial
            pb16 = partial.astype(jnp.bfloat16)
            for c in range(SQ // CH):
                send_buf[(SQ // CH) * b + c, :, :] = pb16[c * CH:(c + 1) * CH, :]

        p1 = []
        for p in peers:
            d = pltpu.make_async_remote_copy(
                src_ref=send_buf.at[p],
                dst_ref=recv_buf.at[my],
                send_sem=s1.at[p],
                recv_sem=r1.at[my],
                device_id=(p,),
                device_id_type=pl.DeviceIdType.MESH,
            )
            d.start()
            p1.append(d)

        for p in peers:
            pltpu.make_async_remote_copy(
                src_ref=send_buf.at[p], dst_ref=recv_buf.at[p],
                send_sem=s1.at[p], recv_sem=r1.at[p],
                device_id=(p,), device_id_type=pl.DeviceIdType.MESH,
            ).wait_recv()

        acc = out_ref[pl.ds(my * CH, CH), :]
        for p in peers:
            acc = acc + recv_buf[p].astype(jnp.float32)
        bcast_buf[:, :] = acc.astype(jnp.bfloat16)
        out_ref[pl.ds(my * CH, CH), :] = acc

        for d in p1:
            d.wait_send()

        p2 = []
        for p in peers:
            d = pltpu.make_async_remote_copy(
                src_ref=bcast_buf,
                dst_ref=gather_buf.at[my],
                send_sem=s2.at[p],
                recv_sem=r2.at[my],
                device_id=(p,),
                device_id_type=pl.DeviceIdType.MESH,
            )
            d.start()
            p2.append(d)

        for p in peers:
            pltpu.make_async_remote_copy(
                src_ref=bcast_buf, dst_ref=gather_buf.at[p],
                send_sem=s2.at[p], recv_sem=r2.at[p],
                device_id=(p,), device_id_type=pl.DeviceIdType.MESH,
            ).wait_recv()
            out_ref[pl.ds(p * CH, CH), :] = gather_buf[p].astype(jnp.float32)

        for d in p2:
            d.wait_send()

    out2d = pl.pallas_call(
        body,
        out_shape=jax.ShapeDtypeStruct((ROWS, DM), jnp.float32),
        in_specs=[pl.BlockSpec(memory_space=pltpu.VMEM)] * 5,
        out_specs=pl.BlockSpec(memory_space=pltpu.VMEM),
        scratch_shapes=[
            pltpu.VMEM((N_DEV, CH, DM), jnp.bfloat16),
            pltpu.VMEM((N_DEV, CH, DM), jnp.bfloat16),
            pltpu.VMEM((N_DEV, CH, DM), jnp.bfloat16),
            pltpu.VMEM((CH, DM), jnp.bfloat16),
            pltpu.SemaphoreType.DMA((N_DEV,)),
            pltpu.SemaphoreType.DMA((N_DEV,)),
            pltpu.SemaphoreType.DMA((N_DEV,)),
            pltpu.SemaphoreType.DMA((N_DEV,)),
        ],
        compiler_params=pltpu.CompilerParams(collective_id=0),
    )(x, Wq, K2, V2, Wo)
    return out2d.reshape(B, SQ, DM)
